# baseline (speedup 1.0000x reference)
import functools
import jax
import jax.numpy as jnp
from jax import lax
from jax.experimental import pallas as pl
from jax.experimental.pallas import tpu as pltpu
from jax.experimental.pallas import tpu_sc as plsc

S = 32768
A = 1024
N = 256
NC = 2
NS = 16
NW = NC * NS
CH_ROWS = 32
CH = CH_ROWS * A
NK = S // (CH_ROWS * NW)
TOT_CH = S // CH_ROWS
TERM_CH0 = TOT_CH - (N // CH_ROWS)
PREV_CH0 = TERM_CH0 - (N // CH_ROWS)
NBUF = 3
TERM_W0 = NW - (N // CH_ROWS)


def _sc_body(x_ref, d2_ref, o_ref, b0, b1, b2, dbuf, isem, osem):
    cidx = lax.axis_index("c")
    sidx = lax.axis_index("s")
    w = sidx * NC + cidx
    bufs = (b0, b1, b2)
    cins = {}
    couts = {}

    def g_of(k):
        return k * NW + w

    def start_in(k):
        b = k % NBUF
        c = pltpu.make_async_copy(
            x_ref.at[pl.ds(g_of(k) * CH, CH)], bufs[b], isem.at[b])
        if k == NK - 1:
            @pl.when(w < TERM_W0)
            def _():
                c.start()
        else:
            c.start()
        cins[k] = c

    def wait_in(k):
        if k == NK - 1:
            @pl.when(w < TERM_W0)
            def _():
                cins[k].wait()
        else:
            cins[k].wait()

    def start_out(k):
        b = k % NBUF
        c = pltpu.make_async_copy(
            bufs[b], o_ref.at[pl.ds(g_of(k) * CH, CH)], osem.at[b])
        if k == NK - 1:
            @pl.when(w < TERM_W0)
            def _():
                c.start()
        else:
            c.start()
        couts[k] = c

    def wait_out(k):
        if k == NK - 1:
            @pl.when(w < TERM_W0)
            def _():
                couts[k].wait()
        else:
            couts[k].wait()

    for k in range(NBUF):
        start_in(k)
    for k in range(NK):
        wait_in(k)
        start_out(k)
        if k + NBUF < NK:
            wait_out(k)
            start_in(k + NBUF)
    for k in range(NK - NBUF, NK):
        wait_out(k)

    wp = jnp.maximum(w - TERM_W0, 0)

    @pl.when(w >= TERM_W0)
    def _term():
        pltpu.sync_copy(x_ref.at[pl.ds((PREV_CH0 + wp) * CH, CH)], b0)
        pltpu.sync_copy(d2_ref.at[pl.ds(wp * CH_ROWS * 16, CH_ROWS * 16)], dbuf)

        def addbody(i, carry):
            off = i * 16
            doff = (i // (A // 16)) * 16
            b0[pl.ds(off, 16)] = b0[pl.ds(off, 16)] + dbuf[pl.ds(doff, 16)]
            return carry

        lax.fori_loop(0, CH // 16, addbody, 0)
        pltpu.sync_copy(b0, o_ref.at[pl.ds((TERM_CH0 + wp) * CH, CH)])


def sc_kernel(expected, drift):
    x_flat = expected.reshape(S * A)
    d2 = jnp.broadcast_to(drift[:, None], (N, 16)).reshape(N * 16)
    mesh = plsc.VectorSubcoreMesh(core_axis_name="c", subcore_axis_name="s")
    body = functools.partial(
        pl.kernel,
        out_type=jax.ShapeDtypeStruct((S * A,), jnp.float32),
        mesh=mesh,
        scratch_types=[
            pltpu.VMEM((CH,), jnp.float32),
            pltpu.VMEM((CH,), jnp.float32),
            pltpu.VMEM((CH,), jnp.float32),
            pltpu.VMEM((CH_ROWS * 16,), jnp.float32),
            pltpu.SemaphoreType.DMA((NBUF,)),
            pltpu.SemaphoreType.DMA((NBUF,)),
        ],
    )(_sc_body)
    out = body(x_flat, d2)
    return out.reshape(S, A)


def kernel(expected, drift):
    return sc_kernel(expected, drift)


# speedup vs baseline: 4.2110x; 4.2110x over previous
import jax
import jax.numpy as jnp
from jax.experimental import pallas as pl
from jax.experimental.pallas import tpu as pltpu

S = 32768
A = 1024
N = 256
B = 2048
M = 5
LAG = 0
NCH = S // B


def _body(x_ref, d_ref, o_ref, *rest):
    bufs = rest[:M]
    isem, osem = rest[M], rest[M + 1]
    cins = [None] * NCH
    couts = [None] * NCH
    waited = set()

    def start_in(i):
        b = i % M
        c = pltpu.make_async_copy(
            x_ref.at[pl.ds(i * B, B), :], bufs[b], isem.at[b])
        c.start()
        cins[i] = c

    for i in range(M):
        start_in(i)
    for i in range(NCH):
        b = i % M
        cins[i].wait()
        if i == NCH - 1:
            bufs[b][B - N:B, :] = bufs[b][B - 2 * N:B - N, :] + d_ref[...]
        c = pltpu.make_async_copy(
            bufs[b], o_ref.at[pl.ds(i * B, B), :], osem.at[b])
        c.start()
        couts[i] = c
        j = i - LAG
        if j >= 0 and j + M < NCH:
            couts[j].wait()
            waited.add(j)
            start_in(j + M)
    for i in range(NCH):
        if i not in waited:
            couts[i].wait()


def kernel(expected, drift):
    drift2d = drift.reshape(N, 1)
    return pl.pallas_call(
        _body,
        in_specs=[
            pl.BlockSpec(memory_space=pltpu.MemorySpace.HBM),
            pl.BlockSpec(memory_space=pltpu.MemorySpace.VMEM),
        ],
        out_specs=pl.BlockSpec(memory_space=pltpu.MemorySpace.HBM),
        out_shape=jax.ShapeDtypeStruct((S, A), expected.dtype),
        scratch_shapes=(
            [pltpu.VMEM((B, A), jnp.float32) for _ in range(M)]
            + [pltpu.SemaphoreType.DMA((M,)), pltpu.SemaphoreType.DMA((M,))]
        ),
    )(expected, drift2d)
